# Initial kernel scaffold; baseline (speedup 1.0000x reference)
import jax
import jax.numpy as jnp
from jax.experimental import pallas as pl


def kernel(inputs, argmax):
    raise NotImplementedError("write your pallas kernel here")


# speedup vs baseline: 3.9822x; 3.9822x over previous
import jax
import jax.numpy as jnp
from jax import lax
from jax.experimental import pallas as pl
from jax.experimental.pallas import tpu as pltpu
from jax.experimental.pallas import tpu_sc as plsc

B, H, W, C = 8, 112, 112, 96
N = B * H * W * C
M = B * (2 * H) * (2 * W) * C

NC, NS, LANES = 2, 16, 16
CHUNK = (2 ** 18) * 7
NCHUNK = M // CHUNK
CHUNKS_PER_CORE = -(-NCHUNK // NC)
ACC = CHUNK + NS * LANES
PER_TILE = N // NS
BLK = 2048
NBLK = PER_TILE // BLK
GRP = 4
NGRP = BLK // (GRP * LANES)
ZERO_WORDS = 2048
ZSLICE = ACC // NS
NZ_FULL = ZSLICE // ZERO_WORDS
Z_TAIL = ZSLICE - NZ_FULL * ZERO_WORDS
OUT_SLICE = CHUNK // NS
BATCH = 128
FLUSH_AT = 64


def _body(idx_hbm, val_hbm, out_hbm,
          buf_i, buf_v, dma_i, dma_v, zbuf, acc,
          isem0, isem1, fsem0, fsem1):
    c = lax.axis_index("c")
    s = lax.axis_index("s")
    in_base = s * PER_TILE
    lane = lax.broadcasted_iota(jnp.int32, (LANES,), 0)
    trash = jnp.int32(CHUNK) + s * LANES + lane
    ucap = jnp.uint32(CHUNK)

    def zb(i, carry):
        zbuf[pl.ds(i * LANES, LANES)] = jnp.zeros((LANES,), jnp.float32)
        return carry

    lax.fori_loop(0, ZERO_WORDS // LANES, zb, 0)

    def issue_flush(row, sem):
        pltpu.sync_copy(dma_v.at[row], acc.at[dma_i.at[row]], add=True)

    def wait_flush(row, sem):
        pass

    def pad_row(row, off, npad):
        for j in range(npad):
            pos = off + (j * 16) + lane
            plsc.store_scatter(dma_i.at[row], [pos], trash, mask=pos < BATCH)

    def in_src(b):
        return (idx_hbm.at[pl.ds(in_base + b * BLK, BLK)],
                val_hbm.at[pl.ds(in_base + b * BLK, BLK)])

    def chunk_body(j, carry):
        chunk_id = j * NC + c

        @pl.when(chunk_id < NCHUNK)
        def _():
            base = chunk_id * CHUNK
            zoff = s * ZSLICE

            def z(k, carry):
                pltpu.sync_copy(zbuf, acc.at[pl.ds(zoff + k * ZERO_WORDS,
                                                   ZERO_WORDS)])
                return carry

            lax.fori_loop(0, NZ_FULL, z, 0)
            pltpu.sync_copy(zbuf.at[pl.ds(0, Z_TAIL)],
                            acc.at[pl.ds(zoff + NZ_FULL * ZERO_WORDS, Z_TAIL)])
            plsc.subcore_barrier()

            src_i0, src_v0 = in_src(0)
            pltpu.async_copy(src_i0, buf_i.at[0], isem0)
            pltpu.async_copy(src_v0, buf_v.at[0], isem0)

            def blk_body(b, carry):
                off, fc = carry
                k = lax.rem(b, 2)
                src_i, src_v = in_src(b)

                @pl.when(k == 0)
                def _():
                    pltpu.make_async_copy(src_i, buf_i.at[0], isem0).wait()
                    pltpu.make_async_copy(src_v, buf_v.at[0], isem0).wait()

                    @pl.when(b + 1 < NBLK)
                    def _():
                        ni, nv = in_src(b + 1)
                        pltpu.async_copy(ni, buf_i.at[1], isem1)
                        pltpu.async_copy(nv, buf_v.at[1], isem1)

                @pl.when(k == 1)
                def _():
                    pltpu.make_async_copy(src_i, buf_i.at[1], isem1).wait()
                    pltpu.make_async_copy(src_v, buf_v.at[1], isem1).wait()

                    @pl.when(b + 1 < NBLK)
                    def _():
                        ni, nv = in_src(b + 1)
                        pltpu.async_copy(ni, buf_i.at[0], isem0)
                        pltpu.async_copy(nv, buf_v.at[0], isem0)

                def grp_body(g, carry):
                    off, fc = carry
                    fk = lax.rem(fc, 2)
                    for u in range(GRP):
                        iv = buf_i[k, pl.ds(g * (GRP * LANES) + u * LANES,
                                            LANES)]
                        vv = buf_v[k, pl.ds(g * (GRP * LANES) + u * LANES,
                                            LANES)]
                        rel = iv - base
                        mask = plsc.bitcast(rel, jnp.uint32) < ucap
                        plsc.store_compressed(dma_i.at[fk, pl.ds(off, LANES)],
                                              rel, mask=mask)
                        plsc.store_compressed(dma_v.at[fk, pl.ds(off, LANES)],
                                              vv, mask=mask)
                        cv = plsc.all_reduce_population_count(mask)
                        off = off + lax.squeeze(lax.slice(cv, (0,), (1,)),
                                                (0,))
                    do_flush = off >= FLUSH_AT

                    @pl.when(do_flush & (fk == 0))
                    def _():
                        pad_row(0, off, 4)
                        issue_flush(0, fsem0)

                        @pl.when(fc >= 1)
                        def _():
                            wait_flush(1, fsem1)

                    @pl.when(do_flush & (fk == 1))
                    def _():
                        pad_row(1, off, 4)
                        issue_flush(1, fsem1)

                        @pl.when(fc >= 1)
                        def _():
                            wait_flush(0, fsem0)

                    off = jnp.where(do_flush, jnp.int32(0), off)
                    fc = jnp.where(do_flush, fc + 1, fc)
                    return (off, fc)

                return lax.fori_loop(0, NGRP, grp_body, (off, fc))

            off, fc = lax.fori_loop(0, NBLK, blk_body,
                                    (jnp.int32(0), jnp.int32(0)))

            fk = lax.rem(fc, 2)
            did = off > 0

            @pl.when(did & (fk == 0))
            def _():
                pad_row(0, off, 8)
                issue_flush(0, fsem0)

            @pl.when(did & (fk == 1))
            def _():
                pad_row(1, off, 8)
                issue_flush(1, fsem1)

            @pl.when((fc >= 1) & (fk == 1))
            def _():
                wait_flush(0, fsem0)

            @pl.when((fc >= 1) & (fk == 0))
            def _():
                wait_flush(1, fsem1)

            @pl.when(did & (fk == 0))
            def _():
                wait_flush(0, fsem0)

            @pl.when(did & (fk == 1))
            def _():
                wait_flush(1, fsem1)

            plsc.subcore_barrier()
            pltpu.sync_copy(acc.at[pl.ds(s * OUT_SLICE, OUT_SLICE)],
                            out_hbm.at[pl.ds(base + s * OUT_SLICE, OUT_SLICE)])
            plsc.subcore_barrier()

        return carry

    lax.fori_loop(0, CHUNKS_PER_CORE, chunk_body, 0)


def kernel(inputs, argmax):
    flat_val = inputs.reshape(-1)
    flat_idx = argmax.reshape(-1).astype(jnp.int32)
    mesh = plsc.VectorSubcoreMesh(core_axis_name="c", subcore_axis_name="s")
    k = pl.kernel(
        _body,
        out_type=jax.ShapeDtypeStruct((M,), jnp.float32),
        mesh=mesh,
        compiler_params=pltpu.CompilerParams(needs_layout_passes=False),
        scratch_types=[
            pltpu.VMEM((2, BLK), jnp.int32),
            pltpu.VMEM((2, BLK), jnp.float32),
            pltpu.VMEM((2, BATCH), jnp.int32),
            pltpu.VMEM((2, BATCH), jnp.float32),
            pltpu.VMEM((ZERO_WORDS,), jnp.float32),
            pltpu.VMEM_SHARED((ACC,), jnp.float32),
            pltpu.SemaphoreType.DMA,
            pltpu.SemaphoreType.DMA,
            pltpu.SemaphoreType.DMA,
            pltpu.SemaphoreType.DMA,
        ],
    )
    out = k(flat_idx, flat_val)
    return out.reshape(B, 2 * H, 2 * W, C)
